# baseline (device time: 4596190 ns/iter reference)
import jax
import jax.numpy as jnp
from jax import lax
from jax.experimental import pallas as pl
from jax.experimental.pallas import tpu as pltpu

M = 8192
D = 2048
CH = 512
NC = M // CH
NZ = 4


def kernel(partial, resid, gamma):
    gamma2 = gamma.reshape(1, D)

    def body(
        x_ref,
        resid_ref,
        gamma_ref,
        out_ref,
        lx,
        re,
        rw,
        se,
        sw,
        ox,
        rxv,
        load_sems,
        resid_sems,
        out_sems,
        sendp_sems, recvp_sems, sends_sems, recvs_sems,
    ):
        my_x = lax.axis_index("x")
        my_y = lax.axis_index("y")
        my_z = lax.axis_index("z")
        has_p = my_z > 0
        has_s = my_z < NZ - 1
        up = jnp.minimum(my_z + 1, NZ - 1)
        dn = jnp.maximum(my_z - 1, 0)

        barrier_sem = pltpu.get_barrier_semaphore()

        @pl.when(has_s)
        def _():
            pl.semaphore_signal(
                barrier_sem, inc=1,
                device_id=(my_x, my_y, up),
                device_id_type=pl.DeviceIdType.MESH,
            )

        @pl.when(has_p)
        def _():
            pl.semaphore_signal(
                barrier_sem, inc=1,
                device_id=(my_x, my_y, dn),
                device_id_type=pl.DeviceIdType.MESH,
            )

        pl.semaphore_wait(barrier_sem, 1)

        @pl.when(jnp.logical_and(has_p, has_s))
        def _():
            pl.semaphore_wait(barrier_sem, 1)

        def rdma_p(slot):
            return pltpu.make_async_remote_copy(
                src_ref=se.at[slot],
                dst_ref=re.at[slot],
                send_sem=sendp_sems.at[slot],
                recv_sem=recvp_sems.at[slot],
                device_id=(my_x, my_y, up),
                device_id_type=pl.DeviceIdType.MESH,
            )

        def rdma_s(slot):
            return pltpu.make_async_remote_copy(
                src_ref=sw.at[slot],
                dst_ref=rw.at[slot],
                send_sem=sends_sems.at[slot],
                recv_sem=recvs_sems.at[slot],
                device_id=(my_x, my_y, dn),
                device_id_type=pl.DeviceIdType.MESH,
            )

        def load_local(i, slot):
            rows = pl.ds(i * CH, CH)
            return pltpu.make_async_copy(
                x_ref.at[0, rows, :], lx.at[slot], load_sems.at[slot]
            )

        def load_resid(i, slot):
            rows = pl.ds(i * CH, CH)
            return pltpu.make_async_copy(
                resid_ref.at[rows, :], rxv.at[slot], resid_sems.at[slot]
            )

        def store_out(i, slot):
            rows = pl.ds(i * CH, CH)
            return pltpu.make_async_copy(
                ox.at[slot], out_ref.at[rows, :], out_sems.at[slot]
            )

        load_local(0, 0).start()
        load_resid(0, 0).start()

        for i in range(NC):
            slot = i % 2

            if i + 1 < NC:
                load_local(i + 1, (i + 1) % 2).start()
                load_resid(i + 1, (i + 1) % 2).start()

            load_local(i, slot).wait()

            @pl.when(has_p)
            def _():
                rdma_p(slot).wait_recv()

            @pl.when(has_s)
            def _():
                if i >= 2:
                    rdma_p(slot).wait_send()
                se[slot] = lx[slot] + jnp.where(has_p, re[slot], 0.0)
                rdma_p(slot).start()

            @pl.when(has_s)
            def _():
                rdma_s(slot).wait_recv()

            @pl.when(has_p)
            def _():
                if i >= 2:
                    rdma_s(slot).wait_send()
                sw[slot] = lx[slot] + jnp.where(has_s, rw[slot], 0.0)
                rdma_s(slot).start()

            load_resid(i, slot).wait()
            y = (
                lx[slot]
                + rxv[slot]
                + jnp.where(has_p, re[slot], 0.0)
                + jnp.where(has_s, rw[slot], 0.0)
            )
            rms = jnp.sqrt(jnp.mean(y * y, axis=-1, keepdims=True) + 1e-6)
            if i >= 2:
                store_out(i - 2, slot).wait()
            ox[slot] = y / rms * gamma_ref[...]
            store_out(i, slot).start()

        for i in (NC - 2, NC - 1):
            slot = i % 2

            @pl.when(has_s)
            def _():
                rdma_p(slot).wait_send()

            @pl.when(has_p)
            def _():
                rdma_s(slot).wait_send()

            store_out(i, slot).wait()

    return pl.pallas_call(
        body,
        out_shape=jax.ShapeDtypeStruct((M, D), jnp.float32),
        in_specs=[
            pl.BlockSpec(memory_space=pl.ANY),
            pl.BlockSpec(memory_space=pl.ANY),
            pl.BlockSpec(memory_space=pltpu.VMEM),
        ],
        out_specs=pl.BlockSpec(memory_space=pl.ANY),
        scratch_shapes=[
            pltpu.VMEM((2, CH, D), jnp.float32),
            pltpu.VMEM((2, CH, D), jnp.float32),
            pltpu.VMEM((2, CH, D), jnp.float32),
            pltpu.VMEM((2, CH, D), jnp.float32),
            pltpu.VMEM((2, CH, D), jnp.float32),
            pltpu.VMEM((2, CH, D), jnp.float32),
            pltpu.VMEM((2, CH, D), jnp.float32),
            pltpu.SemaphoreType.DMA((2,)),
            pltpu.SemaphoreType.DMA((2,)),
            pltpu.SemaphoreType.DMA((2,)),
            pltpu.SemaphoreType.DMA((2,)),
            pltpu.SemaphoreType.DMA((2,)),
            pltpu.SemaphoreType.DMA((2,)),
            pltpu.SemaphoreType.DMA((2,)),
        ],
        compiler_params=pltpu.CompilerParams(
            collective_id=0, vmem_limit_bytes=96 * 1024 * 1024
        ),
    )(partial, resid, gamma2)


# device time: 827870 ns/iter; 5.5518x vs baseline; 5.5518x over previous
import jax
import jax.numpy as jnp
from jax import lax
from jax.experimental import pallas as pl
from jax.experimental.pallas import tpu as pltpu

M = 8192
D = 2048
CH = 256
NC = M // CH
NZ = 4
NSTEP = NC + 3


def kernel(partial, resid, gamma):
    gamma2 = gamma.reshape(1, D)

    def body(
        x_ref,
        resid_ref,
        gamma_ref,
        out_ref,
        lx,
        re,
        rw,
        se,
        sw,
        ox,
        rx,
        load_sems,
        resid_sems,
        out_sems,
        sendp_sems,
        recvp_sems,
        sends_sems,
        recvs_sems,
    ):
        my_x = lax.axis_index("x")
        my_y = lax.axis_index("y")
        my_z = lax.axis_index("z")
        has_p = my_z > 0
        has_s = my_z < NZ - 1
        up = jnp.minimum(my_z + 1, NZ - 1)
        dn = jnp.maximum(my_z - 1, 0)

        barrier_sem = pltpu.get_barrier_semaphore()

        @pl.when(has_s)
        def _():
            pl.semaphore_signal(
                barrier_sem, inc=1,
                device_id=(my_x, my_y, up),
                device_id_type=pl.DeviceIdType.MESH,
            )

        @pl.when(has_p)
        def _():
            pl.semaphore_signal(
                barrier_sem, inc=1,
                device_id=(my_x, my_y, dn),
                device_id_type=pl.DeviceIdType.MESH,
            )

        pl.semaphore_wait(barrier_sem, 1)

        @pl.when(jnp.logical_and(has_p, has_s))
        def _():
            pl.semaphore_wait(barrier_sem, 1)

        def rdma_p(k):
            return pltpu.make_async_remote_copy(
                src_ref=se.at[lax.rem(k, 2)],
                dst_ref=re.at[lax.rem(k, 6)],
                send_sem=sendp_sems.at[lax.rem(k, 2)],
                recv_sem=recvp_sems.at[lax.rem(k, 6)],
                device_id=(my_x, my_y, up),
                device_id_type=pl.DeviceIdType.MESH,
            )

        def rdma_s(k):
            return pltpu.make_async_remote_copy(
                src_ref=sw.at[lax.rem(k, 2)],
                dst_ref=rw.at[lax.rem(k, 6)],
                send_sem=sends_sems.at[lax.rem(k, 2)],
                recv_sem=recvs_sems.at[lax.rem(k, 6)],
                device_id=(my_x, my_y, dn),
                device_id_type=pl.DeviceIdType.MESH,
            )

        def load_local(k):
            rows = pl.ds(k * CH, CH)
            return pltpu.make_async_copy(
                x_ref.at[0, rows, :],
                lx.at[lax.rem(k, 6)],
                load_sems.at[lax.rem(k, 6)],
            )

        def load_resid(k):
            rows = pl.ds(k * CH, CH)
            return pltpu.make_async_copy(
                resid_ref.at[rows, :],
                rx.at[lax.rem(k, 2)],
                resid_sems.at[lax.rem(k, 2)],
            )

        def store_out(k):
            rows = pl.ds(k * CH, CH)
            return pltpu.make_async_copy(
                ox.at[lax.rem(k, 2)],
                out_ref.at[rows, :],
                out_sems.at[lax.rem(k, 2)],
            )

        load_local(0).start()

        def step(t, carry):
            @pl.when(t + 1 <= NC - 1)
            def _():
                load_local(t + 1).start()

            @pl.when(jnp.logical_and(t - 2 >= 0, t - 2 <= NC - 1))
            def _():
                load_resid(t - 2).start()

            @pl.when(t <= NC - 1)
            def _():
                load_local(t).wait()

            jp = t - my_z
            on_p = jnp.logical_and(jp >= 0, jp <= NC - 1)

            @pl.when(jnp.logical_and(on_p, has_p))
            def _():
                rdma_p(jp).wait_recv()

            @pl.when(jnp.logical_and(on_p, has_s))
            def _():
                @pl.when(jp >= 2)
                def _():
                    rdma_p(jp - 2).wait_send()

                se[lax.rem(jp, 2)] = lx[lax.rem(jp, 6)] + jnp.where(
                    has_p, re[lax.rem(jp, 6)], 0.0
                )
                rdma_p(jp).start()

            js = t - 3 + my_z
            on_s = jnp.logical_and(js >= 0, js <= NC - 1)

            @pl.when(jnp.logical_and(on_s, has_s))
            def _():
                rdma_s(js).wait_recv()

            @pl.when(jnp.logical_and(on_s, has_p))
            def _():
                @pl.when(js >= 2)
                def _():
                    rdma_s(js - 2).wait_send()

                sw[lax.rem(js, 2)] = lx[lax.rem(js, 6)] + jnp.where(
                    has_s, rw[lax.rem(js, 6)], 0.0
                )
                rdma_s(js).start()

            c = t - 3

            @pl.when(jnp.logical_and(c >= 0, c <= NC - 1))
            def _():
                load_resid(c).wait()
                y = (
                    lx[lax.rem(c, 6)]
                    + rx[lax.rem(c, 2)]
                    + jnp.where(has_p, re[lax.rem(c, 6)], 0.0)
                    + jnp.where(has_s, rw[lax.rem(c, 6)], 0.0)
                )
                rms = jnp.sqrt(jnp.mean(y * y, axis=-1, keepdims=True) + 1e-6)

                @pl.when(c >= 2)
                def _():
                    store_out(c - 2).wait()

                ox[lax.rem(c, 2)] = y / rms * gamma_ref[...]
                store_out(c).start()

            return carry

        lax.fori_loop(0, NSTEP, step, 0)

        for k in (NC - 2, NC - 1):

            @pl.when(has_s)
            def _():
                rdma_p(k).wait_send()

            @pl.when(has_p)
            def _():
                rdma_s(k).wait_send()

            store_out(k).wait()

    return pl.pallas_call(
        body,
        out_shape=jax.ShapeDtypeStruct((M, D), jnp.float32),
        in_specs=[
            pl.BlockSpec(memory_space=pl.ANY),
            pl.BlockSpec(memory_space=pl.ANY),
            pl.BlockSpec(memory_space=pltpu.VMEM),
        ],
        out_specs=pl.BlockSpec(memory_space=pl.ANY),
        scratch_shapes=[
            pltpu.VMEM((6, CH, D), jnp.float32),
            pltpu.VMEM((6, CH, D), jnp.float32),
            pltpu.VMEM((6, CH, D), jnp.float32),
            pltpu.VMEM((2, CH, D), jnp.float32),
            pltpu.VMEM((2, CH, D), jnp.float32),
            pltpu.VMEM((2, CH, D), jnp.float32),
            pltpu.VMEM((2, CH, D), jnp.float32),
            pltpu.SemaphoreType.DMA((6,)),
            pltpu.SemaphoreType.DMA((2,)),
            pltpu.SemaphoreType.DMA((2,)),
            pltpu.SemaphoreType.DMA((2,)),
            pltpu.SemaphoreType.DMA((6,)),
            pltpu.SemaphoreType.DMA((2,)),
            pltpu.SemaphoreType.DMA((6,)),
        ],
        compiler_params=pltpu.CompilerParams(
            collective_id=0, vmem_limit_bytes=96 * 1024 * 1024
        ),
    )(partial, resid, gamma2)
